# baseline (speedup 1.0000x reference)
import functools

import jax
import jax.numpy as jnp
from jax.experimental import pallas as pl
from jax.experimental.pallas import tpu as pltpu


def _se_hwnc_kernel(x_ref, w1_ref, b1_ref, w2_ref, b2_ref, o_ref, *,
                    hw, unroll):
    S, C = x_ref.shape[1], x_ref.shape[2]

    def add_body(i, acc):
        for u in range(unroll):
            acc = acc + x_ref[i * unroll + u]
        return acc
    acc = jnp.zeros((S, C), jnp.float32)
    acc = jax.lax.fori_loop(0, hw // unroll, add_body, acc)
    for u in range(hw - hw % unroll, hw):
        acc = acc + x_ref[u]
    mean = acc * (1.0 / hw)

    z = jax.lax.dot_general(mean, w1_ref[...], (((1,), (1,)), ((), ())),
                            preferred_element_type=jnp.float32)
    h = jnp.maximum(z + b1_ref[...], 0.0)
    v = jax.lax.dot_general(h, w2_ref[...], (((1,), (1,)), ((), ())),
                            preferred_element_type=jnp.float32)
    s = jnp.clip(v + b2_ref[...] + 3.0, 0.0, 6.0) * (1.0 / 6.0)

    def mul_body(i, _):
        for u in range(unroll):
            j = i * unroll + u
            o_ref[j] = x_ref[j] * s
        return 0
    jax.lax.fori_loop(0, hw // unroll, mul_body, 0)
    for u in range(hw - hw % unroll, hw):
        o_ref[u] = x_ref[u] * s


def kernel(x, w1, b1, w2, b2):
    N, C, H, W = x.shape
    HW = H * W
    Csq = w1.shape[0]

    xt = x.transpose(2, 3, 0, 1).reshape(HW, N, C)
    b1r = b1.reshape(1, Csq)
    b2r = b2.reshape(1, C)

    S = 64
    while N % S:
        S //= 2
    db = x.dtype.itemsize
    cost = pl.CostEstimate(
        flops=2 * N * (2 * C * Csq) + 3 * N * C * HW,
        transcendentals=0,
        bytes_accessed=2 * N * C * HW * db,
    )

    out = pl.pallas_call(
        functools.partial(_se_hwnc_kernel, hw=HW, unroll=28),
        out_shape=jax.ShapeDtypeStruct((HW, N, C), x.dtype),
        grid=(N // S,),
        in_specs=[
            pl.BlockSpec((HW, S, C), lambda n: (0, n, 0)),
            pl.BlockSpec((Csq, C), lambda n: (0, 0)),
            pl.BlockSpec((1, Csq), lambda n: (0, 0)),
            pl.BlockSpec((C, Csq), lambda n: (0, 0)),
            pl.BlockSpec((1, C), lambda n: (0, 0)),
        ],
        out_specs=pl.BlockSpec((HW, S, C), lambda n: (0, n, 0)),
        compiler_params=pltpu.CompilerParams(
            dimension_semantics=("parallel",),
            vmem_limit_bytes=57 << 20,
        ),
        cost_estimate=cost,
    )(xt, w1, b1r, w2, b2r)

    return out.reshape(H, W, N, C).transpose(2, 3, 0, 1)


# speedup vs baseline: 1.0009x; 1.0009x over previous
import functools

import jax
import jax.numpy as jnp
from jax.experimental import pallas as pl
from jax.experimental.pallas import tpu as pltpu


def _se_hwnc_kernel(x_ref, w1_ref, b1_ref, w2_ref, b2_ref, o_ref, *,
                    hw, unroll):
    S, C = x_ref.shape[1], x_ref.shape[2]

    def add_body(i, acc):
        for u in range(unroll):
            acc = acc + x_ref[i * unroll + u]
        return acc
    acc = jnp.zeros((S, C), jnp.float32)
    acc = jax.lax.fori_loop(0, hw // unroll, add_body, acc)
    for u in range(hw - hw % unroll, hw):
        acc = acc + x_ref[u]
    mean = acc * (1.0 / hw)

    z = jax.lax.dot_general(mean, w1_ref[...], (((1,), (1,)), ((), ())),
                            preferred_element_type=jnp.float32)
    h = jnp.maximum(z + b1_ref[...], 0.0)
    v = jax.lax.dot_general(h, w2_ref[...], (((1,), (1,)), ((), ())),
                            preferred_element_type=jnp.float32)
    s = jnp.clip(v + b2_ref[...] + 3.0, 0.0, 6.0) * (1.0 / 6.0)

    def mul_body(i, _):
        for u in range(unroll):
            j = i * unroll + u
            o_ref[j] = x_ref[j] * s
        return 0
    jax.lax.fori_loop(0, hw // unroll, mul_body, 0)
    for u in range(hw - hw % unroll, hw):
        o_ref[u] = x_ref[u] * s


def kernel(x, w1, b1, w2, b2):
    N, C, H, W = x.shape
    HW = H * W
    Csq = w1.shape[0]

    xt = x.transpose(2, 3, 0, 1).reshape(HW, N, C)
    b1r = b1.reshape(1, Csq)
    b2r = b2.reshape(1, C)

    S = 64
    while N % S:
        S //= 2
    db = x.dtype.itemsize
    cost = pl.CostEstimate(
        flops=2 * N * (2 * C * Csq) + 3 * N * C * HW,
        transcendentals=0,
        bytes_accessed=2 * N * C * HW * db,
    )

    out = pl.pallas_call(
        functools.partial(_se_hwnc_kernel, hw=HW, unroll=14),
        out_shape=jax.ShapeDtypeStruct((HW, N, C), x.dtype),
        grid=(N // S,),
        in_specs=[
            pl.BlockSpec((HW, S, C), lambda n: (0, n, 0)),
            pl.BlockSpec((Csq, C), lambda n: (0, 0)),
            pl.BlockSpec((1, Csq), lambda n: (0, 0)),
            pl.BlockSpec((C, Csq), lambda n: (0, 0)),
            pl.BlockSpec((1, C), lambda n: (0, 0)),
        ],
        out_specs=pl.BlockSpec((HW, S, C), lambda n: (0, n, 0)),
        compiler_params=pltpu.CompilerParams(
            dimension_semantics=("parallel",),
            vmem_limit_bytes=57 << 20,
        ),
        cost_estimate=cost,
    )(xt, w1, b1r, w2, b2r)

    return out.reshape(H, W, N, C).transpose(2, 3, 0, 1)
